# baseline (speedup 1.0000x reference)
import functools

import jax
import jax.numpy as jnp
from jax import lax
from jax.experimental import pallas as pl
from jax.experimental.pallas import tpu as pltpu
from jax.experimental.pallas import tpu_sc as plsc

NUM_USERS = 1000000
NUM_ITEMS = 1000000
EMB_SIZE = 64
BATCH = 16384

_info = plsc.get_sparse_core_info()
_NC, _NS = _info.num_cores, _info.num_subcores
_NW = _NC * _NS
_BPW = BATCH // _NW


def _gather4(U_e, V_e, users, items, neg_users, neg_items,
             u_out, negu_out, v_out, negi_out,
             idx_v, rows_v, sem):
    wid = lax.axis_index("s") * _NC + lax.axis_index("c")
    base = wid * _BPW
    jobs = (
        (U_e, users, u_out),
        (U_e, neg_users, negu_out),
        (V_e, items, v_out),
        (V_e, neg_items, negi_out),
    )
    for tbl, idx_hbm, out_hbm in jobs:
        pltpu.sync_copy(idx_hbm.at[pl.ds(base, _BPW)], idx_v)
        pltpu.async_copy(tbl.at[idx_v], rows_v, sem).wait()
        pltpu.sync_copy(rows_v, out_hbm.at[pl.ds(base, _BPW)])


_mesh = plsc.VectorSubcoreMesh(core_axis_name="c", subcore_axis_name="s")

_kernel_call = functools.partial(
    pl.kernel,
    out_type=[jax.ShapeDtypeStruct((BATCH, EMB_SIZE), jnp.float32)] * 4,
    mesh=_mesh,
    scratch_types=[
        pltpu.VMEM((_BPW,), jnp.int32),
        pltpu.VMEM((_BPW, EMB_SIZE), jnp.float32),
        pltpu.SemaphoreType.DMA,
    ],
    compiler_params=pltpu.CompilerParams(use_tc_tiling_on_sc=False),
)(_gather4)


@jax.jit
def kernel(U_e, V_e, users, items, neg_users, neg_items):
    users = users.astype(jnp.int32)
    items = items.astype(jnp.int32)
    neg_users = neg_users.astype(jnp.int32)
    neg_items = neg_items.astype(jnp.int32)
    u_e, neg_u_e, v_e, neg_v_e = _kernel_call(
        U_e, V_e, users, items, neg_users, neg_items)
    return (u_e, neg_u_e, v_e, neg_v_e)


# speedup vs baseline: 2.2033x; 2.2033x over previous
import functools

import jax
import jax.numpy as jnp
from jax import lax
from jax.experimental import pallas as pl
from jax.experimental.pallas import tpu as pltpu
from jax.experimental.pallas import tpu_sc as plsc

EMB = 64
BATCH = 16384
NROW = 1000000

_info = plsc.get_sparse_core_info()
_NC, _NS = _info.num_cores, _info.num_subcores
_NW = _NC * _NS

_CW = 1024
_NFULL = NROW // _CW
_REM_LO = _NFULL * _CW
_REM_OWNER = _NFULL % _NW
_TAIL_LO = 999936
_NBIN = 32
_CAP = 4096
_SROWS = BATCH + _NW * 16


def _scan_job(idx_hbm, wid, idx_v, hidx, hr):
    pltpu.sync_copy(idx_hbm, idx_v)
    lanes = lax.iota(jnp.int32, 16)

    def body(n, cnt):
        vec = idx_v[pl.ds(n * 16, 16)]
        c = lax.shift_right_logical(vec, 10)
        m = (c & (_NW - 1)) == wid
        pop = plsc.all_reduce_population_count(m)[0]
        plsc.store_compressed(hidx.at[pl.ds(cnt, 16)], vec, mask=m)
        plsc.store_compressed(hr.at[pl.ds(cnt, 16)], n * 16 + lanes, mask=m)
        return cnt + pop

    return lax.fori_loop(0, BATCH // 16, body, jnp.int32(0), unroll=False)


def _sort_job(cnt, hidx, hr, sidx, sr, start, off):
    lanes = lax.iota(jnp.int32, 16)
    m0 = lanes == 0
    for o in range(_NBIN):
        off[o] = jnp.int32(0)

    def hist(g, carry):
        valid = g * 16 + lanes < cnt
        ov = jnp.where(valid, lax.shift_right_logical(
            hidx[pl.ds(g * 16, 16)], 15), _NBIN - 1)
        for l in range(16):
            o = ov[l]
            off[o] = off[o] + 1
        return carry

    ngrp = (cnt + 15) // 16
    lax.fori_loop(0, ngrp, hist, jnp.int32(0), unroll=False)

    s = jnp.int32(0)
    for o in range(_NBIN):
        t = off[o]
        start[o] = s
        off[o] = s
        s = s + t

    def place(g, carry):
        valid = g * 16 + lanes < cnt
        iv = hidx[pl.ds(g * 16, 16)]
        rv = hr[pl.ds(g * 16, 16)]
        ov = jnp.where(valid, lax.shift_right_logical(iv, 15), _NBIN - 1)
        for l in range(16):
            o = ov[l]
            p = off[o]
            off[o] = p + 1
            plsc.store_scatter(sidx, [jnp.full((16,), p)],
                               jnp.full((16,), iv[l]), mask=m0)
            plsc.store_scatter(sr, [jnp.full((16,), p)],
                               jnp.full((16,), rv[l]), mask=m0)
        return carry

    lax.fori_loop(0, ngrp, place, jnp.int32(0), unroll=False)


def _emit_run(chunk_v, S, lo, s, e, sidx, sr, brow, rb_v, dump_rows, lanes):
    hsel = lax.shift_right_logical(lanes, 2)
    jsub = lanes & 3

    def group(g, carry):
        p = s + g * 16
        n = e - p
        valid16 = lanes < n
        rv = jnp.where(valid16, sr[pl.ds(p, 16)], dump_rows)
        rb_v[...] = rv
        for hb in range(4):
            hitv = hb * 4 + hsel
            mask = hitv < n
            colv = jnp.where(
                mask, plsc.load_gather(sidx, [p + hitv]) - lo, 0)
            for jb in range(16):
                jv = jb * 4 + jsub
                x = plsc.load_gather(chunk_v, [jv, colv])
                plsc.store_scatter(brow, [hitv, jv], x, mask=mask)
        pltpu.sync_copy(brow, S.at[rb_v])
        return carry

    lax.fori_loop(0, (e - s + 15) // 16, group, jnp.int32(0), unroll=False)


def _k1_body(U_t, V_t, U_tail, V_tail, users, items, neg_users, neg_items,
             SA, SB, SC_, SD,
             idx_v, hidx, hr, sidxA, srA, sidxB, srB,
             chunk_v, brow, rb_v, startA, offA, startB, offB):
    wid = lax.axis_index("s") * _NC + lax.axis_index("c")
    lanes = lax.iota(jnp.int32, 16)
    dump_rows = BATCH + wid * 16 + lanes

    for tbl, tail, jobs in (
            (U_t, U_tail, ((users, SA, sidxA, srA, startA, offA),
                           (neg_users, SB, sidxB, srB, startB, offB))),
            (V_t, V_tail, ((items, SC_, sidxA, srA, startA, offA),
                           (neg_items, SD, sidxB, srB, startB, offB)))):
        for idx_hbm, _S, sidx, sr, start, off in jobs:
            cnt = _scan_job(idx_hbm, wid, idx_v, hidx, hr)
            _sort_job(cnt, hidx, hr, sidx, sr, start, off)

        nfull = (_NFULL - 1 - wid) // _NW + 1

        def chunk_body(o, carry):
            c = o * _NW + wid
            lo = pl.multiple_of(c * _CW, _CW)
            pltpu.sync_copy(tbl.at[:, pl.ds(lo, _CW)], chunk_v)
            for _idx, S, sidx, sr, start, off in jobs:
                _emit_run(chunk_v, S, lo, start[o], off[o], sidx, sr,
                          brow, rb_v, dump_rows, lanes)
            return carry

        lax.fori_loop(0, nfull, chunk_body, jnp.int32(0), unroll=False)

        @pl.when(wid == _REM_OWNER)
        def _():
            o = jnp.int32(_NFULL // _NW)
            lo = pl.multiple_of(jnp.int32(_REM_LO), 512)
            pltpu.sync_copy(tbl.at[:, pl.ds(lo, 512)],
                            chunk_v.at[:, pl.ds(0, 512)])
            pltpu.sync_copy(tail, chunk_v.at[:, pl.ds(512, 128)])
            for _idx, S, sidx, sr, start, off in jobs:
                _emit_run(chunk_v, S, lo, start[o], off[o], sidx, sr,
                          brow, rb_v, dump_rows, lanes)


def _k2_body(SA, SB, SC_, SD, uo, nuo, vo, nio, blk_v, obuf_v):
    wid = lax.axis_index("s") * _NC + lax.axis_index("c")
    lanes = lax.iota(jnp.int32, 16)

    for S, out in ((SA, uo), (SB, nuo), (SC_, vo), (SD, nio)):
        for q in range(BATCH // 128 // _NW):
            b = wid * (BATCH // 128 // _NW) + q
            rlo = pl.multiple_of(b * 128, 128)
            pltpu.sync_copy(S.at[pl.ds(rlo, 128), :], blk_v)

            def rloop(r, carry):
                for jg in range(4):
                    jv = lanes + jg * 16
                    x = plsc.load_gather(blk_v, [jnp.full((16,), r), jv])
                    plsc.store_scatter(obuf_v, [jv, jnp.full((16,), r)], x)
                return carry

            lax.fori_loop(0, 128, rloop, jnp.int32(0), unroll=False)
            pltpu.sync_copy(obuf_v, out.at[:, pl.ds(rlo, 128)])


_mesh = plsc.VectorSubcoreMesh(core_axis_name="c", subcore_axis_name="s")

_k1 = functools.partial(
    pl.kernel,
    out_type=[jax.ShapeDtypeStruct((_SROWS, 128), jnp.float32)] * 4,
    mesh=_mesh,
    compiler_params=pltpu.CompilerParams(needs_layout_passes=False),
    scratch_types=[
        pltpu.VMEM((BATCH,), jnp.int32),
        pltpu.VMEM((_CAP + 16,), jnp.int32),
        pltpu.VMEM((_CAP + 16,), jnp.int32),
        pltpu.VMEM((_CAP + 16,), jnp.int32),
        pltpu.VMEM((_CAP + 16,), jnp.int32),
        pltpu.VMEM((_CAP + 16,), jnp.int32),
        pltpu.VMEM((_CAP + 16,), jnp.int32),
        pltpu.VMEM((EMB, _CW), jnp.float32),
        pltpu.VMEM((16, 128), jnp.float32),
        pltpu.VMEM((16,), jnp.int32),
        pltpu.SMEM((_NBIN,), jnp.int32),
        pltpu.SMEM((_NBIN,), jnp.int32),
        pltpu.SMEM((_NBIN,), jnp.int32),
        pltpu.SMEM((_NBIN,), jnp.int32),
    ],
)(_k1_body)

_k2 = functools.partial(
    pl.kernel,
    out_type=[jax.ShapeDtypeStruct((EMB, BATCH), jnp.float32)] * 4,
    mesh=_mesh,
    compiler_params=pltpu.CompilerParams(needs_layout_passes=False),
    scratch_types=[
        pltpu.VMEM((128, 128), jnp.float32),
        pltpu.VMEM((EMB, 128), jnp.float32),
    ],
)(_k2_body)


@jax.jit
def kernel(U_e, V_e, users, items, neg_users, neg_items):
    users = users.astype(jnp.int32)
    items = items.astype(jnp.int32)
    neg_users = neg_users.astype(jnp.int32)
    neg_items = neg_items.astype(jnp.int32)
    U_t, V_t = U_e.T, V_e.T
    U_tail = jnp.pad(U_t[:, _TAIL_LO:], ((0, 0), (0, 128 - (NROW - _TAIL_LO))))
    V_tail = jnp.pad(V_t[:, _TAIL_LO:], ((0, 0), (0, 128 - (NROW - _TAIL_LO))))
    SA, SB, SC_, SD = _k1(U_t, V_t, U_tail, V_tail,
                          users, items, neg_users, neg_items)
    u_t, negu_t, v_t, negi_t = _k2(SA, SB, SC_, SD)
    return (u_t.T, negu_t.T, v_t.T, negi_t.T)


# speedup vs baseline: 2.8724x; 1.3037x over previous
import functools

import jax
import jax.numpy as jnp
from jax import lax
from jax.experimental import pallas as pl
from jax.experimental.pallas import tpu as pltpu
from jax.experimental.pallas import tpu_sc as plsc

EMB = 64
BATCH = 16384
NROW = 1000000

_info = plsc.get_sparse_core_info()
_NC, _NS = _info.num_cores, _info.num_subcores
_NW = _NC * _NS

_CW = 512
_NFULL = NROW // _CW
_REM_OWNER = _NFULL % _NW
_TAIL_LO = 999936
_OSHIFT = 14
_NBIN = 64
_CAP = 4096
_SROWS = BATCH + _NW * 16


def _scan_job(idx_hbm, wid, idx_v, hidx, hr):
    pltpu.sync_copy(idx_hbm, idx_v)
    lanes = lax.iota(jnp.int32, 16)

    def body(n, cnt):
        vec = idx_v[pl.ds(n * 16, 16)]
        c = lax.shift_right_logical(vec, 9)
        m = (c & (_NW - 1)) == wid
        pop = plsc.all_reduce_population_count(m)[0]
        plsc.store_compressed(hidx.at[pl.ds(cnt, 16)], vec, mask=m)
        plsc.store_compressed(hr.at[pl.ds(cnt, 16)], n * 16 + lanes, mask=m)
        return cnt + pop

    return lax.fori_loop(0, BATCH // 16, body, jnp.int32(0), unroll=False)


def _sort_job(cnt, hidx, hr, sidx, sr, start, off):
    lanes = lax.iota(jnp.int32, 16)
    m0 = lanes == 0
    for o in range(_NBIN):
        off[o] = jnp.int32(0)

    def hist(g, carry):
        valid = g * 16 + lanes < cnt
        ov = jnp.where(valid, lax.shift_right_logical(
            hidx[pl.ds(g * 16, 16)], _OSHIFT), _NBIN - 1)
        for l in range(16):
            o = ov[l]
            off[o] = off[o] + 1
        return carry

    ngrp = (cnt + 15) // 16
    lax.fori_loop(0, ngrp, hist, jnp.int32(0), unroll=False)

    s = jnp.int32(0)
    for o in range(_NBIN):
        t = off[o]
        start[o] = s
        off[o] = s
        s = s + t

    def place(g, carry):
        valid = g * 16 + lanes < cnt
        iv = hidx[pl.ds(g * 16, 16)]
        rv = hr[pl.ds(g * 16, 16)]
        ov = jnp.where(valid, lax.shift_right_logical(iv, _OSHIFT), _NBIN - 1)
        for l in range(16):
            o = ov[l]
            p = off[o]
            off[o] = p + 1
            plsc.store_scatter(sidx, [jnp.full((16,), p)],
                               jnp.full((16,), iv[l]), mask=m0)
            plsc.store_scatter(sr, [jnp.full((16,), p)],
                               jnp.full((16,), rv[l]), mask=m0)
        return carry

    lax.fori_loop(0, ngrp, place, jnp.int32(0), unroll=False)


def _emit_run(chunk_v, S, lo, s, e, sidx, sr, brow, rb_v, dump_rows, lanes):
    hsel = lax.shift_right_logical(lanes, 2)
    jsub = lanes & 3

    def group(g, carry):
        p = s + g * 16
        n = e - p
        valid16 = lanes < n
        rv = jnp.where(valid16, sr[pl.ds(p, 16)], dump_rows)
        rb_v[...] = rv
        for hb in range(4):
            hitv = hb * 4 + hsel
            mask = hitv < n
            colv = jnp.where(
                mask, plsc.load_gather(sidx, [p + hitv]) - lo, 0)
            for jb in range(16):
                jv = jb * 4 + jsub
                x = plsc.load_gather(chunk_v, [jv, colv])
                plsc.store_scatter(brow, [hitv, jv], x, mask=mask)
        pltpu.sync_copy(brow, S.at[rb_v])
        return carry

    lax.fori_loop(0, (e - s + 15) // 16, group, jnp.int32(0), unroll=False)


def _k1_body(U_t, V_t, U_tail, V_tail, users, items, neg_users, neg_items,
             SA, SB, SC_, SD,
             idx_v, hidx, hr, sidxA, srA, sidxB, srB,
             chunk_a, chunk_b, brow, rb_v, semA, semB,
             startA, offA, startB, offB):
    wid = lax.axis_index("s") * _NC + lax.axis_index("c")
    lanes = lax.iota(jnp.int32, 16)
    dump_rows = BATCH + wid * 16 + lanes

    for tbl, tail, jobs in (
            (U_t, U_tail, ((users, SA, sidxA, srA, startA, offA),
                           (neg_users, SB, sidxB, srB, startB, offB))),
            (V_t, V_tail, ((items, SC_, sidxA, srA, startA, offA),
                           (neg_items, SD, sidxB, srB, startB, offB)))):
        for idx_hbm, _S, sidx, sr, start, off in jobs:
            cnt = _scan_job(idx_hbm, wid, idx_v, hidx, hr)
            _sort_job(cnt, hidx, hr, sidx, sr, start, off)

        nfull = (_NFULL - 1 - wid) // _NW + 1

        def fire(o, buf, sem):
            @pl.when(o < nfull)
            def _():
                lo = pl.multiple_of((o * _NW + wid) * _CW, _CW)
                pltpu.async_copy(tbl.at[:, pl.ds(lo, _CW)], buf, sem)

        def process(o, buf, sem):
            @pl.when(o < nfull)
            def _():
                pltpu.make_async_copy(
                    tbl.at[:, pl.ds(0, _CW)], buf, sem).wait()
                lo = pl.multiple_of((o * _NW + wid) * _CW, _CW)
                for _idx, S, sidx, sr, start, off in jobs:
                    _emit_run(buf, S, lo, start[o], off[o], sidx, sr,
                              brow, rb_v, dump_rows, lanes)
                fire(o + 2, buf, sem)

        fire(jnp.int32(0), chunk_a, semA)
        fire(jnp.int32(1), chunk_b, semB)

        def chunk_pair(m, carry):
            process(2 * m, chunk_a, semA)
            process(2 * m + 1, chunk_b, semB)
            return carry

        lax.fori_loop(0, (_NFULL // _NW + 2) // 2, chunk_pair, jnp.int32(0),
                      unroll=False)

        @pl.when(wid == _REM_OWNER)
        def _():
            o = jnp.int32(_NFULL // _NW)
            lo = jnp.int32(_TAIL_LO)
            pltpu.sync_copy(tail, chunk_a.at[:, pl.ds(0, 128)])
            for _idx, S, sidx, sr, start, off in jobs:
                _emit_run(chunk_a, S, lo, start[o], off[o], sidx, sr,
                          brow, rb_v, dump_rows, lanes)


def _k2_body(SA, SB, SC_, SD, uo, nuo, vo, nio, blk_v, obuf_v):
    wid = lax.axis_index("s") * _NC + lax.axis_index("c")
    lanes = lax.iota(jnp.int32, 16)

    for S, out in ((SA, uo), (SB, nuo), (SC_, vo), (SD, nio)):
        for q in range(BATCH // 128 // _NW):
            b = wid * (BATCH // 128 // _NW) + q
            rlo = pl.multiple_of(b * 128, 128)
            pltpu.sync_copy(S.at[pl.ds(rlo, 128), :], blk_v)

            rsub = lanes & 3
            jq = lax.shift_right_logical(lanes, 2)

            def rloop(r0, carry):
                rv4 = r0 * 4 + rsub
                for jb in range(16):
                    jv4 = jb * 4 + jq
                    x = plsc.load_gather(blk_v, [rv4, jv4])
                    plsc.store_scatter(obuf_v, [jv4, rv4], x)
                return carry

            lax.fori_loop(0, 32, rloop, jnp.int32(0), unroll=False)
            pltpu.sync_copy(obuf_v, out.at[:, pl.ds(rlo, 128)])


_mesh = plsc.VectorSubcoreMesh(core_axis_name="c", subcore_axis_name="s")

_k1 = functools.partial(
    pl.kernel,
    out_type=[jax.ShapeDtypeStruct((_SROWS, 128), jnp.float32)] * 4,
    mesh=_mesh,
    compiler_params=pltpu.CompilerParams(needs_layout_passes=False),
    scratch_types=[
        pltpu.VMEM((BATCH,), jnp.int32),
        pltpu.VMEM((_CAP + 16,), jnp.int32),
        pltpu.VMEM((_CAP + 16,), jnp.int32),
        pltpu.VMEM((_CAP + 16,), jnp.int32),
        pltpu.VMEM((_CAP + 16,), jnp.int32),
        pltpu.VMEM((_CAP + 16,), jnp.int32),
        pltpu.VMEM((_CAP + 16,), jnp.int32),
        pltpu.VMEM((EMB, _CW), jnp.float32),
        pltpu.VMEM((EMB, _CW), jnp.float32),
        pltpu.VMEM((16, 128), jnp.float32),
        pltpu.VMEM((16,), jnp.int32),
        pltpu.SemaphoreType.DMA,
        pltpu.SemaphoreType.DMA,
        pltpu.SMEM((_NBIN,), jnp.int32),
        pltpu.SMEM((_NBIN,), jnp.int32),
        pltpu.SMEM((_NBIN,), jnp.int32),
        pltpu.SMEM((_NBIN,), jnp.int32),
    ],
)(_k1_body)

_k2 = functools.partial(
    pl.kernel,
    out_type=[jax.ShapeDtypeStruct((EMB, BATCH), jnp.float32)] * 4,
    mesh=_mesh,
    compiler_params=pltpu.CompilerParams(needs_layout_passes=False),
    scratch_types=[
        pltpu.VMEM((128, 128), jnp.float32),
        pltpu.VMEM((EMB, 128), jnp.float32),
    ],
)(_k2_body)


@jax.jit
def kernel(U_e, V_e, users, items, neg_users, neg_items):
    users = users.astype(jnp.int32)
    items = items.astype(jnp.int32)
    neg_users = neg_users.astype(jnp.int32)
    neg_items = neg_items.astype(jnp.int32)
    U_t, V_t = U_e.T, V_e.T
    U_tail = jnp.pad(U_t[:, _TAIL_LO:], ((0, 0), (0, 128 - (NROW - _TAIL_LO))))
    V_tail = jnp.pad(V_t[:, _TAIL_LO:], ((0, 0), (0, 128 - (NROW - _TAIL_LO))))
    SA, SB, SC_, SD = _k1(U_t, V_t, U_tail, V_tail,
                          users, items, neg_users, neg_items)
    u_t, negu_t, v_t, negi_t = _k2(SA, SB, SC_, SD)
    return (u_t.T, negu_t.T, v_t.T, negi_t.T)


# speedup vs baseline: 2.9412x; 1.0239x over previous
import functools

import jax
import jax.numpy as jnp
from jax import lax
from jax.experimental import pallas as pl
from jax.experimental.pallas import tpu as pltpu
from jax.experimental.pallas import tpu_sc as plsc

EMB = 64
BATCH = 16384
NROW = 1000000

_info = plsc.get_sparse_core_info()
_NC, _NS = _info.num_cores, _info.num_subcores
_NW = _NC * _NS

_CW = 512
_NFULL = NROW // _CW
_REM_OWNER = _NFULL % _NW
_TAIL_LO = 999936
_OSHIFT = 14
_NBIN = 64
_CAP = 4096
_SROWS = BATCH + _NW * 16


def _scan_job(idx_hbm, wid, idx_v, hidx, hr):
    pltpu.sync_copy(idx_hbm, idx_v)
    lanes = lax.iota(jnp.int32, 16)

    def body(n, cnt):
        vec = idx_v[pl.ds(n * 16, 16)]
        c = lax.shift_right_logical(vec, 9)
        m = (c & (_NW - 1)) == wid
        pop = plsc.all_reduce_population_count(m)[0]
        plsc.store_compressed(hidx.at[pl.ds(cnt, 16)], vec, mask=m)
        plsc.store_compressed(hr.at[pl.ds(cnt, 16)], n * 16 + lanes, mask=m)
        return cnt + pop

    return lax.fori_loop(0, BATCH // 16, body, jnp.int32(0), unroll=False)


def _sort_job(cnt, hidx, hr, sidx, sr, start, off):
    lanes = lax.iota(jnp.int32, 16)
    m0 = lanes == 0
    for o in range(_NBIN):
        off[o] = jnp.int32(0)

    def hist(g, carry):
        valid = g * 16 + lanes < cnt
        ov = jnp.where(valid, lax.shift_right_logical(
            hidx[pl.ds(g * 16, 16)], _OSHIFT), _NBIN - 1)
        for l in range(16):
            o = ov[l]
            off[o] = off[o] + 1
        return carry

    ngrp = (cnt + 15) // 16
    lax.fori_loop(0, ngrp, hist, jnp.int32(0), unroll=False)

    s = jnp.int32(0)
    for o in range(_NBIN):
        t = off[o]
        start[o] = s
        off[o] = s
        s = s + t

    def place(g, carry):
        valid = g * 16 + lanes < cnt
        iv = hidx[pl.ds(g * 16, 16)]
        rv = hr[pl.ds(g * 16, 16)]
        ov = jnp.where(valid, lax.shift_right_logical(iv, _OSHIFT), _NBIN - 1)
        for l in range(16):
            o = ov[l]
            p = off[o]
            off[o] = p + 1
            plsc.store_scatter(sidx, [jnp.full((16,), p)],
                               jnp.full((16,), iv[l]), mask=m0)
            plsc.store_scatter(sr, [jnp.full((16,), p)],
                               jnp.full((16,), rv[l]), mask=m0)
        return carry

    lax.fori_loop(0, ngrp, place, jnp.int32(0), unroll=False)


def _emit_run(chunk_v, S, lo, s, e, sidx, sr, bufs, dump_rows, lanes):
    hsel = lax.shift_right_logical(lanes, 2)
    jsub = lanes & 3

    def one_group(g, brow, rb_v, sem):
        p = s + g * 16
        n = e - p

        @pl.when(n > 0)
        def _():
            pltpu.make_async_copy(S.at[pl.ds(0, 16), :], brow, sem).wait()
            valid16 = lanes < n
            rv = jnp.where(valid16, sr[pl.ds(p, 16)], dump_rows)
            rb_v[...] = rv
            def hb_loop(hb, carry):
                hitv = hb * 4 + hsel
                mask = hitv < n
                colv = jnp.where(
                    mask, plsc.load_gather(sidx, [p + hitv]) - lo, 0)

                def jb_loop(jb, carry2):
                    jv = jb * 4 + jsub
                    x = plsc.load_gather(chunk_v, [jv, colv])
                    plsc.store_scatter(brow, [hitv, jv], x, mask=mask)
                    return carry2

                lax.fori_loop(0, 16, jb_loop, jnp.int32(0), unroll=False)
                return carry

            lax.fori_loop(0, 4, hb_loop, jnp.int32(0), unroll=False)
            pltpu.async_copy(brow, S.at[rb_v], sem)

    def group_pair(m, carry):
        one_group(2 * m, *bufs[0])
        one_group(2 * m + 1, *bufs[1])
        return carry

    lax.fori_loop(0, (e - s + 31) // 32, group_pair, jnp.int32(0),
                  unroll=False)


def _k1_body(U_t, V_t, U_tail, V_tail, users, items, neg_users, neg_items,
             SA, SB, SC_, SD,
             idx_v, hidx, hr, sidxA, srA, sidxB, srB,
             chunk_a, chunk_b, browA0, rbA0, browA1, rbA1,
             browB0, rbB0, browB1, rbB1, semA, semB,
             fsA0, fsA1, fsB0, fsB1,
             startA, offA, startB, offB):
    wid = lax.axis_index("s") * _NC + lax.axis_index("c")
    lanes = lax.iota(jnp.int32, 16)
    dump_rows = BATCH + wid * 16 + lanes
    bufsA = ((browA0, rbA0, fsA0), (browA1, rbA1, fsA1))
    bufsB = ((browB0, rbB0, fsB0), (browB1, rbB1, fsB1))
    all_bufs = bufsA + bufsB
    for brow, _rb, fsem in all_bufs:
        pltpu.async_copy(SA.at[pl.ds(0, 16), :], brow, fsem)

    for tbl, tail, jobs in (
            (U_t, U_tail, ((users, SA, sidxA, srA, startA, offA, bufsA),
                           (neg_users, SB, sidxB, srB, startB, offB, bufsB))),
            (V_t, V_tail, ((items, SC_, sidxA, srA, startA, offA, bufsA),
                           (neg_items, SD, sidxB, srB, startB, offB,
                            bufsB)))):
        for idx_hbm, _S, sidx, sr, start, off, _bufs in jobs:
            cnt = _scan_job(idx_hbm, wid, idx_v, hidx, hr)
            _sort_job(cnt, hidx, hr, sidx, sr, start, off)

        nfull = (_NFULL - 1 - wid) // _NW + 1

        def fire(o, buf, sem):
            @pl.when(o < nfull)
            def _():
                lo = pl.multiple_of((o * _NW + wid) * _CW, _CW)
                pltpu.async_copy(tbl.at[:, pl.ds(lo, _CW)], buf, sem)

        def process(o, buf, sem):
            @pl.when(o < nfull)
            def _():
                pltpu.make_async_copy(
                    tbl.at[:, pl.ds(0, _CW)], buf, sem).wait()
                lo = pl.multiple_of((o * _NW + wid) * _CW, _CW)
                for _idx, S, sidx, sr, start, off, jbufs in jobs:
                    _emit_run(buf, S, lo, start[o], off[o], sidx, sr,
                              jbufs, dump_rows, lanes)
                fire(o + 2, buf, sem)

        fire(jnp.int32(0), chunk_a, semA)
        fire(jnp.int32(1), chunk_b, semB)

        def chunk_pair(m, carry):
            process(2 * m, chunk_a, semA)
            process(2 * m + 1, chunk_b, semB)
            return carry

        lax.fori_loop(0, (_NFULL // _NW + 2) // 2, chunk_pair, jnp.int32(0),
                      unroll=False)

        @pl.when(wid == _REM_OWNER)
        def _():
            o = jnp.int32(_NFULL // _NW)
            lo = jnp.int32(_TAIL_LO)
            pltpu.sync_copy(tail, chunk_a.at[:, pl.ds(0, 128)])
            for _idx, S, sidx, sr, start, off, jbufs in jobs:
                _emit_run(chunk_a, S, lo, start[o], off[o], sidx, sr,
                          jbufs, dump_rows, lanes)

    _k1_drain(SA, all_bufs)


def _k1_drain(SA, all_bufs):
    for brow, _rb, fsem in all_bufs:
        pltpu.make_async_copy(SA.at[pl.ds(0, 16), :], brow, fsem).wait()


def _k2_body(SA, SB, SC_, SD, uo, nuo, vo, nio, blk_v, obuf_v):
    wid = lax.axis_index("s") * _NC + lax.axis_index("c")
    lanes = lax.iota(jnp.int32, 16)

    for S, out in ((SA, uo), (SB, nuo), (SC_, vo), (SD, nio)):
        for q in range(BATCH // 128 // _NW):
            b = wid * (BATCH // 128 // _NW) + q
            rlo = pl.multiple_of(b * 128, 128)
            pltpu.sync_copy(S.at[pl.ds(rlo, 128), :], blk_v)

            rsub = lanes & 3
            jq = lax.shift_right_logical(lanes, 2)

            def rloop(r0, carry):
                rv4 = r0 * 4 + rsub
                for jb in range(16):
                    jv4 = jb * 4 + jq
                    x = plsc.load_gather(blk_v, [rv4, jv4])
                    plsc.store_scatter(obuf_v, [jv4, rv4], x)
                return carry

            lax.fori_loop(0, 32, rloop, jnp.int32(0), unroll=False)
            pltpu.sync_copy(obuf_v, out.at[:, pl.ds(rlo, 128)])


_mesh = plsc.VectorSubcoreMesh(core_axis_name="c", subcore_axis_name="s")

_k1 = functools.partial(
    pl.kernel,
    out_type=[jax.ShapeDtypeStruct((_SROWS, 128), jnp.float32)] * 4,
    mesh=_mesh,
    compiler_params=pltpu.CompilerParams(needs_layout_passes=False),
    scratch_types=[
        pltpu.VMEM((BATCH,), jnp.int32),
        pltpu.VMEM((_CAP + 16,), jnp.int32),
        pltpu.VMEM((_CAP + 16,), jnp.int32),
        pltpu.VMEM((_CAP + 16,), jnp.int32),
        pltpu.VMEM((_CAP + 16,), jnp.int32),
        pltpu.VMEM((_CAP + 16,), jnp.int32),
        pltpu.VMEM((_CAP + 16,), jnp.int32),
        pltpu.VMEM((EMB, _CW), jnp.float32),
        pltpu.VMEM((EMB, _CW), jnp.float32),
        pltpu.VMEM((16, 128), jnp.float32),
        pltpu.VMEM((16,), jnp.int32),
        pltpu.VMEM((16, 128), jnp.float32),
        pltpu.VMEM((16,), jnp.int32),
        pltpu.VMEM((16, 128), jnp.float32),
        pltpu.VMEM((16,), jnp.int32),
        pltpu.VMEM((16, 128), jnp.float32),
        pltpu.VMEM((16,), jnp.int32),
        pltpu.SemaphoreType.DMA,
        pltpu.SemaphoreType.DMA,
        pltpu.SemaphoreType.DMA,
        pltpu.SemaphoreType.DMA,
        pltpu.SemaphoreType.DMA,
        pltpu.SemaphoreType.DMA,
        pltpu.SMEM((_NBIN,), jnp.int32),
        pltpu.SMEM((_NBIN,), jnp.int32),
        pltpu.SMEM((_NBIN,), jnp.int32),
        pltpu.SMEM((_NBIN,), jnp.int32),
    ],
)(_k1_body)

_k2 = functools.partial(
    pl.kernel,
    out_type=[jax.ShapeDtypeStruct((EMB, BATCH), jnp.float32)] * 4,
    mesh=_mesh,
    compiler_params=pltpu.CompilerParams(needs_layout_passes=False),
    scratch_types=[
        pltpu.VMEM((128, 128), jnp.float32),
        pltpu.VMEM((EMB, 128), jnp.float32),
    ],
)(_k2_body)


@jax.jit
def kernel(U_e, V_e, users, items, neg_users, neg_items):
    users = users.astype(jnp.int32)
    items = items.astype(jnp.int32)
    neg_users = neg_users.astype(jnp.int32)
    neg_items = neg_items.astype(jnp.int32)
    U_t, V_t = U_e.T, V_e.T
    U_tail = jnp.pad(U_t[:, _TAIL_LO:], ((0, 0), (0, 128 - (NROW - _TAIL_LO))))
    V_tail = jnp.pad(V_t[:, _TAIL_LO:], ((0, 0), (0, 128 - (NROW - _TAIL_LO))))
    SA, SB, SC_, SD = _k1(U_t, V_t, U_tail, V_tail,
                          users, items, neg_users, neg_items)
    u_t, negu_t, v_t, negi_t = _k2(SA, SB, SC_, SD)
    return (u_t.T, negu_t.T, v_t.T, negi_t.T)


# speedup vs baseline: 3.1355x; 1.0660x over previous
import functools

import jax
import jax.numpy as jnp
from jax import lax
from jax.experimental import pallas as pl
from jax.experimental.pallas import tpu as pltpu
from jax.experimental.pallas import tpu_sc as plsc

EMB = 64
BATCH = 16384
NROW = 1000000

_info = plsc.get_sparse_core_info()
_NC, _NS = _info.num_cores, _info.num_subcores
_NW = _NC * _NS

_CW = 512
_NFULL = NROW // _CW
_REM_OWNER = _NFULL % _NW
_TAIL_LO = 999936
_OSHIFT = 14
_NBIN = 64
_CAP = 4096
_SROWS = BATCH + _NW * 16


def _scan_job(idx_hbm, wid, idx_v, hidx, hr):
    pltpu.sync_copy(idx_hbm, idx_v)
    lanes = lax.iota(jnp.int32, 16)

    def body(n, cnt):
        for h in range(2):
            base = n * 32 + h * 16
            vec = idx_v[pl.ds(base, 16)]
            c = lax.shift_right_logical(vec, 9)
            m = (c & (_NW - 1)) == wid
            pop = plsc.all_reduce_population_count(m)[0]
            plsc.store_compressed(hidx.at[pl.ds(cnt, 16)], vec, mask=m)
            plsc.store_compressed(hr.at[pl.ds(cnt, 16)], base + lanes, mask=m)
            cnt = cnt + pop
        return cnt

    return lax.fori_loop(0, BATCH // 32, body, jnp.int32(0), unroll=False)


def _sort_job(cnt, hidx, hr, sidx, sr, start, off):
    lanes = lax.iota(jnp.int32, 16)
    m0 = lanes == 0
    for o in range(_NBIN):
        off[o] = jnp.int32(0)

    def hist(g, carry):
        valid = g * 16 + lanes < cnt
        ov = jnp.where(valid, lax.shift_right_logical(
            hidx[pl.ds(g * 16, 16)], _OSHIFT), _NBIN - 1)
        for l in range(16):
            o = ov[l]
            off[o] = off[o] + 1
        return carry

    ngrp = (cnt + 15) // 16
    lax.fori_loop(0, ngrp, hist, jnp.int32(0), unroll=False)

    s = jnp.int32(0)
    for o in range(_NBIN):
        t = off[o]
        start[o] = s
        off[o] = s
        s = s + t

    def place(g, carry):
        valid = g * 16 + lanes < cnt
        iv = hidx[pl.ds(g * 16, 16)]
        rv = hr[pl.ds(g * 16, 16)]
        ov = jnp.where(valid, lax.shift_right_logical(iv, _OSHIFT), _NBIN - 1)
        for l in range(16):
            o = ov[l]
            p = off[o]
            off[o] = p + 1
            plsc.store_scatter(sidx, [jnp.full((16,), p)],
                               jnp.full((16,), iv[l]), mask=m0)
            plsc.store_scatter(sr, [jnp.full((16,), p)],
                               jnp.full((16,), rv[l]), mask=m0)
        return carry

    lax.fori_loop(0, ngrp, place, jnp.int32(0), unroll=False)


def _emit_run(chunk_v, S, lo, s, e, sidx, sr, bufs, dump_rows, lanes):
    hsel = lax.shift_right_logical(lanes, 2)
    jsub = lanes & 3

    def one_group(g, brow, rb_v, sem):
        p = s + g * 16
        n = e - p

        @pl.when(n > 0)
        def _():
            pltpu.make_async_copy(S.at[pl.ds(0, 16), :], brow, sem).wait()
            valid16 = lanes < n
            rv = jnp.where(valid16, sr[pl.ds(p, 16)], dump_rows)
            rb_v[...] = rv
            def hb_loop(hb, carry):
                hitv = hb * 4 + hsel
                mask = hitv < n
                colv = jnp.where(
                    mask, plsc.load_gather(sidx, [p + hitv]) - lo, 0)

                def jb_loop(jb, carry2):
                    jv = jb * 4 + jsub
                    x = plsc.load_gather(chunk_v, [jv, colv])
                    plsc.store_scatter(brow, [hitv, jv], x, mask=mask)
                    return carry2

                lax.fori_loop(0, 16, jb_loop, jnp.int32(0), unroll=False)
                return carry

            lax.fori_loop(0, 4, hb_loop, jnp.int32(0), unroll=False)
            pltpu.async_copy(brow, S.at[rb_v], sem)

    def group_pair(m, carry):
        one_group(2 * m, *bufs[0])
        one_group(2 * m + 1, *bufs[1])
        return carry

    lax.fori_loop(0, (e - s + 31) // 32, group_pair, jnp.int32(0),
                  unroll=False)


def _k1_body(U_t, V_t, U_tail, V_tail, users, items, neg_users, neg_items,
             SA, SB, SC_, SD,
             idx_v, hidx, hr, sidxA, srA, sidxB, srB,
             chunk_a, chunk_b, browA0, rbA0, browA1, rbA1,
             browB0, rbB0, browB1, rbB1, semA, semB,
             fsA0, fsA1, fsB0, fsB1,
             startA, offA, startB, offB):
    wid = lax.axis_index("s") * _NC + lax.axis_index("c")
    lanes = lax.iota(jnp.int32, 16)
    dump_rows = BATCH + wid * 16 + lanes
    bufsA = ((browA0, rbA0, fsA0), (browA1, rbA1, fsA1))
    bufsB = ((browB0, rbB0, fsB0), (browB1, rbB1, fsB1))
    all_bufs = bufsA + bufsB
    for brow, _rb, fsem in all_bufs:
        pltpu.async_copy(SA.at[pl.ds(0, 16), :], brow, fsem)

    for tbl, tail, jobs in (
            (U_t, U_tail, ((users, SA, sidxA, srA, startA, offA, bufsA),
                           (neg_users, SB, sidxB, srB, startB, offB, bufsB))),
            (V_t, V_tail, ((items, SC_, sidxA, srA, startA, offA, bufsA),
                           (neg_items, SD, sidxB, srB, startB, offB,
                            bufsB)))):
        for idx_hbm, _S, sidx, sr, start, off, _bufs in jobs:
            cnt = _scan_job(idx_hbm, wid, idx_v, hidx, hr)
            _sort_job(cnt, hidx, hr, sidx, sr, start, off)

        nfull = (_NFULL - 1 - wid) // _NW + 1

        def fire(o, buf, sem):
            @pl.when(o < nfull)
            def _():
                lo = pl.multiple_of((o * _NW + wid) * _CW, _CW)
                pltpu.async_copy(tbl.at[:, pl.ds(lo, _CW)], buf, sem)

        def process(o, buf, sem):
            @pl.when(o < nfull)
            def _():
                pltpu.make_async_copy(
                    tbl.at[:, pl.ds(0, _CW)], buf, sem).wait()
                lo = pl.multiple_of((o * _NW + wid) * _CW, _CW)
                for _idx, S, sidx, sr, start, off, jbufs in jobs:
                    _emit_run(buf, S, lo, start[o], off[o], sidx, sr,
                              jbufs, dump_rows, lanes)
                fire(o + 2, buf, sem)

        fire(jnp.int32(0), chunk_a, semA)
        fire(jnp.int32(1), chunk_b, semB)

        def chunk_pair(m, carry):
            process(2 * m, chunk_a, semA)
            process(2 * m + 1, chunk_b, semB)
            return carry

        lax.fori_loop(0, (_NFULL // _NW + 2) // 2, chunk_pair, jnp.int32(0),
                      unroll=False)

        @pl.when(wid == _REM_OWNER)
        def _():
            o = jnp.int32(_NFULL // _NW)
            lo = jnp.int32(_TAIL_LO)
            pltpu.sync_copy(tail, chunk_a.at[:, pl.ds(0, 128)])
            for _idx, S, sidx, sr, start, off, jbufs in jobs:
                _emit_run(chunk_a, S, lo, start[o], off[o], sidx, sr,
                          jbufs, dump_rows, lanes)

    _k1_drain(SA, all_bufs)


def _k1_drain(SA, all_bufs):
    for brow, _rb, fsem in all_bufs:
        pltpu.make_async_copy(SA.at[pl.ds(0, 16), :], brow, fsem).wait()


def _k2_body(SA, SB, SC_, SD, uo, nuo, vo, nio,
             blk_a, blk_b, obuf_a, obuf_b, bsa, bsb, osa, osb):
    wid = lax.axis_index("s") * _NC + lax.axis_index("c")
    lanes = lax.iota(jnp.int32, 16)
    rsub = lanes & 3
    jq = lax.shift_right_logical(lanes, 2)

    tasks = []
    for S, out in ((SA, uo), (SB, nuo), (SC_, vo), (SD, nio)):
        for q in range(BATCH // 128 // _NW):
            tasks.append((S, out, q))
    blks = ((blk_a, bsa), (blk_b, bsb))
    obufs = ((obuf_a, osa), (obuf_b, osb))

    def rlo_of(S_out_q):
        _S, _out, q = S_out_q
        return pl.multiple_of((wid * (BATCH // 128 // _NW) + q) * 128, 128)

    pltpu.async_copy(tasks[0][0].at[pl.ds(rlo_of(tasks[0]), 128), :],
                     blk_a, bsa)
    pltpu.async_copy(SA.at[pl.ds(0, EMB), :], obuf_a, osa)
    pltpu.async_copy(SA.at[pl.ds(0, EMB), :], obuf_b, osb)

    for t, (S, out, q) in enumerate(tasks):
        blk_v, bsem = blks[t % 2]
        obuf_v, osem = obufs[t % 2]
        rlo = rlo_of(tasks[t])
        pltpu.make_async_copy(S.at[pl.ds(0, 128), :], blk_v, bsem).wait()
        if t + 1 < len(tasks):
            nS, _nout, _nq = tasks[t + 1]
            nblk, nbsem = blks[(t + 1) % 2]
            pltpu.async_copy(nS.at[pl.ds(rlo_of(tasks[t + 1]), 128), :],
                             nblk, nbsem)
        pltpu.make_async_copy(SA.at[pl.ds(0, EMB), :], obuf_v, osem).wait()

        def rloop(r0, carry):
            rv4 = r0 * 4 + rsub
            for jb in range(16):
                jv4 = jb * 4 + jq
                x = plsc.load_gather(blk_v, [rv4, jv4])
                plsc.store_scatter(obuf_v, [jv4, rv4], x)
            return carry

        lax.fori_loop(0, 32, rloop, jnp.int32(0), unroll=False)
        pltpu.async_copy(obuf_v, out.at[:, pl.ds(rlo, 128)], osem)

    for obuf_v, osem in obufs:
        pltpu.make_async_copy(SA.at[pl.ds(0, EMB), :], obuf_v, osem).wait()


_mesh = plsc.VectorSubcoreMesh(core_axis_name="c", subcore_axis_name="s")

_k1 = functools.partial(
    pl.kernel,
    out_type=[jax.ShapeDtypeStruct((_SROWS, 128), jnp.float32)] * 4,
    mesh=_mesh,
    compiler_params=pltpu.CompilerParams(needs_layout_passes=False),
    scratch_types=[
        pltpu.VMEM((BATCH,), jnp.int32),
        pltpu.VMEM((_CAP + 16,), jnp.int32),
        pltpu.VMEM((_CAP + 16,), jnp.int32),
        pltpu.VMEM((_CAP + 16,), jnp.int32),
        pltpu.VMEM((_CAP + 16,), jnp.int32),
        pltpu.VMEM((_CAP + 16,), jnp.int32),
        pltpu.VMEM((_CAP + 16,), jnp.int32),
        pltpu.VMEM((EMB, _CW), jnp.float32),
        pltpu.VMEM((EMB, _CW), jnp.float32),
        pltpu.VMEM((16, 128), jnp.float32),
        pltpu.VMEM((16,), jnp.int32),
        pltpu.VMEM((16, 128), jnp.float32),
        pltpu.VMEM((16,), jnp.int32),
        pltpu.VMEM((16, 128), jnp.float32),
        pltpu.VMEM((16,), jnp.int32),
        pltpu.VMEM((16, 128), jnp.float32),
        pltpu.VMEM((16,), jnp.int32),
        pltpu.SemaphoreType.DMA,
        pltpu.SemaphoreType.DMA,
        pltpu.SemaphoreType.DMA,
        pltpu.SemaphoreType.DMA,
        pltpu.SemaphoreType.DMA,
        pltpu.SemaphoreType.DMA,
        pltpu.SMEM((_NBIN,), jnp.int32),
        pltpu.SMEM((_NBIN,), jnp.int32),
        pltpu.SMEM((_NBIN,), jnp.int32),
        pltpu.SMEM((_NBIN,), jnp.int32),
    ],
)(_k1_body)

_k2 = functools.partial(
    pl.kernel,
    out_type=[jax.ShapeDtypeStruct((EMB, BATCH), jnp.float32)] * 4,
    mesh=_mesh,
    compiler_params=pltpu.CompilerParams(needs_layout_passes=False),
    scratch_types=[
        pltpu.VMEM((128, 128), jnp.float32),
        pltpu.VMEM((128, 128), jnp.float32),
        pltpu.VMEM((EMB, 128), jnp.float32),
        pltpu.VMEM((EMB, 128), jnp.float32),
        pltpu.SemaphoreType.DMA,
        pltpu.SemaphoreType.DMA,
        pltpu.SemaphoreType.DMA,
        pltpu.SemaphoreType.DMA,
    ],
)(_k2_body)


@jax.jit
def kernel(U_e, V_e, users, items, neg_users, neg_items):
    users = users.astype(jnp.int32)
    items = items.astype(jnp.int32)
    neg_users = neg_users.astype(jnp.int32)
    neg_items = neg_items.astype(jnp.int32)
    U_t, V_t = U_e.T, V_e.T
    U_tail = jnp.pad(U_t[:, _TAIL_LO:], ((0, 0), (0, 128 - (NROW - _TAIL_LO))))
    V_tail = jnp.pad(V_t[:, _TAIL_LO:], ((0, 0), (0, 128 - (NROW - _TAIL_LO))))
    SA, SB, SC_, SD = _k1(U_t, V_t, U_tail, V_tail,
                          users, items, neg_users, neg_items)
    u_t, negu_t, v_t, negi_t = _k2(SA, SB, SC_, SD)
    return (u_t.T, negu_t.T, v_t.T, negi_t.T)
